# baseline (speedup 1.0000x reference)
import functools

import jax
import jax.numpy as jnp
from jax import lax
from jax.experimental import pallas as pl
from jax.experimental.pallas import tpu as pltpu
from jax.experimental.pallas import tpu_sc as plsc

DIM = 128
NC = 2
NS = 16
NW = NC * NS
CHUNK = 128
NBUF = 4
NBUF_O = 4
LOOKAHEAD = 3
L = 16


def _gather_body(table, sidx, tidx, out_s, out_t, idx_v, rows_v, pk_v,
                 gsem, wsem):
    wid = lax.axis_index("s") * NC + lax.axis_index("c")
    n_rows_idx = sidx.shape[0]
    rows_per_w = n_rows_idx // NW
    per_w = rows_per_w * CHUNK
    base = wid * rows_per_w
    nch = 2 * rows_per_w

    pltpu.sync_copy(sidx.at[pl.ds(base, rows_per_w)],
                    idx_v.at[pl.ds(0, rows_per_w)])
    pltpu.sync_copy(tidx.at[pl.ds(base, rows_per_w)],
                    idx_v.at[pl.ds(rows_per_w, rows_per_w)])

    def fire_gather(k):
        return pltpu.async_copy(
            table.at[idx_v.at[k]],
            rows_v.at[pl.ds((k % NBUF) * CHUNK, CHUNK)],
            gsem,
        )

    def out_slice(k):
        out = out_s if k < rows_per_w else out_t
        c = k % rows_per_w
        off = pl.multiple_of((wid * per_w + c * CHUNK) // 2, CHUNK // 2)
        return out.at[pl.ds(off, CHUNK // 2)]

    def pack_chunk(k):
        src = (k % NBUF) * CHUNK
        dst = (k % NBUF_O) * (CHUNK // 2)

        def pair_body(p, _):
            for g in range(DIM // L):
                a = rows_v[src + 2 * p, pl.ds(g * L, L)]
                b = rows_v[src + 2 * p + 1, pl.ds(g * L, L)]
                ai = lax.bitcast_convert_type(a, jnp.int32)
                bi = lax.bitcast_convert_type(b, jnp.int32)
                lo = lax.shift_right_logical(ai + 32768, 16)
                hi = (bi + 32768) & jnp.int32(-65536)
                pk_v[dst + p, pl.ds(g * L, L)] = lo | hi
            return 0

        lax.fori_loop(0, CHUNK // 2, pair_body, 0, unroll=2)

    gh = [None] * nch
    wh = [None] * nch
    w_drained = [False] * nch
    for f in range(min(LOOKAHEAD, nch)):
        gh[f] = fire_gather(f)
    for k in range(nch):
        f = k + LOOKAHEAD
        if f < nch:
            gh[f] = fire_gather(f)
        gh[k].wait()
        if k >= NBUF_O:
            wh[k - NBUF_O].wait()
            w_drained[k - NBUF_O] = True
        pack_chunk(k)
        wh[k] = pltpu.async_copy(
            pk_v.at[pl.ds((k % NBUF_O) * (CHUNK // 2), CHUNK // 2)],
            out_slice(k), wsem)
    for k in range(nch):
        if not w_drained[k]:
            wh[k].wait()


def _sc_gather(Eemb, sidx, tidx):
    B = sidx.shape[0] * CHUNK
    rows_per_w = sidx.shape[0] // NW
    mesh = plsc.VectorSubcoreMesh(core_axis_name="c", subcore_axis_name="s")
    f = functools.partial(
        pl.kernel,
        mesh=mesh,
        out_type=[
            jax.ShapeDtypeStruct((B // 2, DIM), jnp.int32),
            jax.ShapeDtypeStruct((B // 2, DIM), jnp.int32),
        ],
        scratch_types=[
            pltpu.VMEM((2 * rows_per_w, CHUNK), jnp.int32),
            pltpu.VMEM((NBUF * CHUNK, DIM), jnp.float32),
            pltpu.VMEM((NBUF_O * (CHUNK // 2), DIM), jnp.int32),
            pltpu.SemaphoreType.DMA,
            pltpu.SemaphoreType.DMA,
        ],
    )(_gather_body)
    return f(Eemb, sidx, tidx)


def _run_one(pk_ref, we_ref, wo_ref, w, o_ref):
    blk = pk_ref.shape[0]
    xi = pk_ref[:]
    xe = lax.bitcast_convert_type(lax.shift_left(xi, 16), jnp.float32)
    xo = lax.bitcast_convert_type(xi & jnp.int32(-65536), jnp.float32)
    nse = jnp.sum(xe * xe, axis=1, keepdims=True)
    nso = jnp.sum(xo * xo, axis=1, keepdims=True)
    ce = we_ref[:].reshape(blk, 1) * jnp.where(nse > 1.0, lax.rsqrt(nse), 1.0)
    co = wo_ref[:].reshape(blk, 1) * jnp.where(nso > 1.0, lax.rsqrt(nso), 1.0)
    o_ref[:, 0:DIM] = jnp.dot(xe, w, preferred_element_type=jnp.float32) * ce
    o_ref[:, DIM:2 * DIM] = (
        jnp.dot(xo, w, preferred_element_type=jnp.float32) * co)


def _tc_body(spk_ref, tpk_ref, swe_ref, swo_ref, twe_ref, two_ref,
             w_ref, so_ref, to_ref):
    w = w_ref[:]
    _run_one(spk_ref, swe_ref, swo_ref, w, so_ref)
    _run_one(tpk_ref, twe_ref, two_ref, w, to_ref)


def _tc_apply(s_pk, t_pk, swe, swo, twe, two, W, blk=2048):
    B2 = s_pk.shape[0]
    grid = (B2 // blk,)
    row_spec = pl.BlockSpec((blk, DIM), lambda i: (i, 0))
    w_spec = pl.BlockSpec((blk,), lambda i: (i,))
    out_spec = pl.BlockSpec((blk, 2 * DIM), lambda i: (i, 0))
    return pl.pallas_call(
        _tc_body,
        grid=grid,
        in_specs=[row_spec, row_spec, w_spec, w_spec, w_spec, w_spec,
                  pl.BlockSpec((DIM, DIM), lambda i: (0, 0))],
        out_specs=[out_spec, out_spec],
        out_shape=[jax.ShapeDtypeStruct((B2, 2 * DIM), jnp.float32)] * 2,
        compiler_params=pltpu.CompilerParams(
            dimension_semantics=("parallel",)),
    )(s_pk, t_pk, swe, swo, twe, two, W)


def kernel(S_in, T_in, anc, s_weight, t_weight, Eemb, W):
    B = S_in.shape[0]
    sidx = S_in.astype(jnp.int32).reshape(B // CHUNK, CHUNK)
    tidx = T_in.astype(jnp.int32).reshape(B // CHUNK, CHUNK)
    s_pk, t_pk = _sc_gather(Eemb, sidx, tidx)
    S2, T2 = _tc_apply(s_pk, t_pk,
                       s_weight[0::2], s_weight[1::2],
                       t_weight[0::2], t_weight[1::2], W)
    return (S2.reshape(B, DIM), T2.reshape(B, DIM))
